# baseline (speedup 1.0000x reference)
import jax
import jax.numpy as jnp
from jax.experimental import pallas as pl
from jax.experimental.pallas import tpu as pltpu

_B, _S, _DIN, _E, _G, _C = 16, 4096, 8, 64, 128, 8
_R = _B * _DIN
_CHP = 512
_CH = 256
_NOUT = _S // _CH
_PAD = 0.0


_DN0 = (((0,), (0,)), ((), ()))


def _h_from_packed(xn2, w1, b1r, w2, b2r):
    h1 = jnp.concatenate(
        [jax.lax.dot_general(xn2[_DIN * b:_DIN * (b + 1), :], w1, _DN0)
         for b in range(_B)], axis=0)
    h = jnp.maximum(h1 + b1r, 0.0)
    return jnp.maximum(h @ w2 + b2r, 0.0)


def _prep_phase(x2_ref, lrep_ref, len3_ref, w1_ref, b1r_ref, w2_ref,
                b2r_ref, wg_ref, bgr_ref, mr_s, svr_s, gfv_s):
    f32 = jnp.float32
    x2 = x2_ref[...]
    lrep = lrep_ref[...]
    il = jax.lax.broadcasted_iota(jnp.int32, (_R, _S), 1)
    mf = (il < lrep).astype(f32)
    cnt = jnp.maximum(jnp.sum(mf) * (1.0 / _DIN), 1.0)

    rd = jax.lax.broadcasted_iota(jnp.int32, (_DIN, _R), 1)
    dd = jax.lax.broadcasted_iota(jnp.int32, (_DIN, _R), 0)
    m8 = (jax.lax.rem(rd, _DIN) == dd).astype(f32)
    rr = jax.lax.broadcasted_iota(jnp.int32, (_R, _DIN), 0)
    dc = jax.lax.broadcasted_iota(jnp.int32, (_R, _DIN), 1)
    p8 = (jax.lax.rem(rr, _DIN) == dc).astype(f32)

    hi = jax.lax.Precision.HIGHEST
    dn = (((1,), (0,)), ((), ()))
    sum_rows = jnp.sum(x2 * mf, axis=1, keepdims=True)
    mean8 = jax.lax.dot_general(m8, sum_rows, dn, precision=hi) / cnt
    mean_r = jax.lax.dot_general(p8, mean8, dn, precision=hi)
    xc = x2 - mean_r
    sq_rows = jnp.sum((xc * xc) * mf, axis=1, keepdims=True)
    var8 = jax.lax.dot_general(m8, sq_rows, dn, precision=hi) / cnt
    sv8 = jnp.sqrt(var8 + 1e-6)
    sv_r = jax.lax.dot_general(p8, sv8, dn, precision=hi)
    mr_s[...] = mean_r
    svr_s[...] = sv_r
    xn2 = xc / sv_r

    len3 = len3_ref[...]
    w1 = w1_ref[...]
    b1r = b1r_ref[...]
    w2 = w2_ref[...]
    b2r = b2r_ref[...]
    wg = wg_ref[...]
    bgr = bgr_ref[...]
    leng = jnp.broadcast_to(len3, (_B, _CHP, _G))
    igc = jax.lax.broadcasted_iota(jnp.int32, (_B, _CHP, _G), 1)
    gfv = jnp.full((_B, 1, _G), -jnp.inf, dtype=f32)
    for c in range(_S // _CHP):
        h = _h_from_packed(xn2[:, _CHP * c:_CHP * (c + 1)],
                           w1, b1r, w2, b2r)
        g = jnp.maximum(h @ wg + bgr, 0.0)
        maskg = (igc + _CHP * c) < leng
        g3 = jnp.where(maskg, g.reshape(_B, _CHP, _G), -jnp.inf)
        gfv = jnp.maximum(gfv, jnp.max(g3, axis=1, keepdims=True))
    gfv_s[...] = gfv


def _out_phase(k, x2c_ref, len3_ref, wseg_ref, bsegr_ref, w1_ref, b1r_ref,
               w2_ref, b2r_ref, logits_ref, labels_ref, scores_ref,
               cat_ref, mr_s, svr_s, gfv_s):
    base = (k - 1) * _CH
    len3 = len3_ref[...]

    n = _B * _CH
    xn2c = (x2c_ref[...] - mr_s[...]) / svr_s[...]
    h = _h_from_packed(xn2c, w1_ref[...], b1r_ref[...],
                       w2_ref[...], b2r_ref[...])

    ie = jax.lax.broadcasted_iota(jnp.int32, (_B, _CH, _E), 1) + base
    maske = ie < jnp.broadcast_to(len3, (_B, _CH, _E))
    h3 = jnp.where(maske, h.reshape(_B, _CH, _E), _PAD)

    ig = jax.lax.broadcasted_iota(jnp.int32, (_B, _CH, _G), 1) + base
    maskg = ig < jnp.broadcast_to(len3, (_B, _CH, _G))
    gfv3 = jnp.broadcast_to(gfv_s[...], (_B, _CH, _G))
    gfv3 = jnp.where(maskg, gfv3, _PAD)

    cat3 = jnp.concatenate([h3, gfv3], axis=2)
    cat_ref[...] = cat3

    logits = cat3.reshape(n, _E + _G) @ wseg_ref[...] + bsegr_ref[...]
    ic = jax.lax.broadcasted_iota(jnp.int32, (_B, _CH, _C), 1) + base
    maskc = ic < jnp.broadcast_to(len3, (_B, _CH, _C))
    logits2 = jnp.where(maskc.reshape(n, _C), logits, _PAD)
    logits_ref[...] = logits2.reshape(_B, _CH, _C)

    lt = jnp.transpose(logits2)
    m = jnp.max(lt, axis=0, keepdims=True)
    ssum = jnp.sum(jnp.exp(lt - m), axis=0, keepdims=True)
    scores = 1.0 / ssum

    cidx = jax.lax.broadcasted_iota(jnp.int32, (_C, n), 0)
    cand = jnp.where(lt == m, cidx, _C)
    labels = jnp.min(cand, axis=0, keepdims=True)
    labels = jnp.where(jnp.isnan(scores), -1, labels)

    def _rows(v):
        return jnp.concatenate(
            [v[:, _CH * b:_CH * (b + 1)] for b in range(_B)], axis=0)

    scores_ref[...] = _rows(scores)
    labels_ref[...] = _rows(labels)


def _fused_kernel(x2_ref, x2c_ref, lrep_ref, len3_ref, w1_ref, b1r_ref,
                  w2_ref, b2r_ref, wg_ref, bgr_ref, wseg_ref, bsegr_ref,
                  logits_ref, labels_ref, scores_ref, cat_ref,
                  mr_s, svr_s, gfv_s):
    k = pl.program_id(0)

    @pl.when(k == 0)
    def _():
        _prep_phase(x2_ref, lrep_ref, len3_ref, w1_ref, b1r_ref, w2_ref,
                    b2r_ref, wg_ref, bgr_ref, mr_s, svr_s, gfv_s)

    @pl.when(k > 0)
    def _():
        _out_phase(k, x2c_ref, len3_ref, wseg_ref, bsegr_ref, w1_ref,
                   b1r_ref, w2_ref, b2r_ref, logits_ref, labels_ref,
                   scores_ref, cat_ref, mr_s, svr_s, gfv_s)


def kernel(x, lengths, W1, b1, W2, b2, Wg, bg, Wseg, bseg):
    f32 = jnp.float32
    x2 = x.transpose(0, 2, 1).reshape(_R, _S)
    lrep = jnp.repeat(lengths.astype(jnp.int32), _DIN).reshape(_R, 1)
    len3 = lengths.astype(jnp.int32).reshape(_B, 1, 1)
    b1r = b1.reshape(1, _E)
    b2r = b2.reshape(1, _E)
    bgr = bg.reshape(1, _G)
    bsegr = bseg.reshape(1, _C)

    def _ochunk(k):
        kk = jnp.maximum(k - 1, 0)
        return kk

    logits, labels, scores, cat = pl.pallas_call(
        _fused_kernel,
        grid=(1 + _NOUT,),
        in_specs=[
            pl.BlockSpec((_R, _S), lambda k: (0, 0)),
            pl.BlockSpec((_R, _CH), lambda k: (0, _ochunk(k))),
            pl.BlockSpec((_R, 1), lambda k: (0, 0)),
            pl.BlockSpec((_B, 1, 1), lambda k: (0, 0, 0)),
            pl.BlockSpec((_DIN, _E), lambda k: (0, 0)),
            pl.BlockSpec((1, _E), lambda k: (0, 0)),
            pl.BlockSpec((_E, _E), lambda k: (0, 0)),
            pl.BlockSpec((1, _E), lambda k: (0, 0)),
            pl.BlockSpec((_E, _G), lambda k: (0, 0)),
            pl.BlockSpec((1, _G), lambda k: (0, 0)),
            pl.BlockSpec((_E + _G, _C), lambda k: (0, 0)),
            pl.BlockSpec((1, _C), lambda k: (0, 0)),
        ],
        out_specs=[
            pl.BlockSpec((_B, _CH, _C), lambda k: (0, _ochunk(k), 0)),
            pl.BlockSpec((_B, _CH), lambda k: (0, _ochunk(k))),
            pl.BlockSpec((_B, _CH), lambda k: (0, _ochunk(k))),
            pl.BlockSpec((_B, _CH, _E + _G), lambda k: (0, _ochunk(k), 0)),
        ],
        out_shape=[
            jax.ShapeDtypeStruct((_B, _S, _C), f32),
            jax.ShapeDtypeStruct((_B, _S), jnp.int32),
            jax.ShapeDtypeStruct((_B, _S), f32),
            jax.ShapeDtypeStruct((_B, _S, _E + _G), f32),
        ],
        scratch_shapes=[
            pltpu.VMEM((_R, 1), f32),
            pltpu.VMEM((_R, 1), f32),
            pltpu.VMEM((_B, 1, _G), f32),
        ],
    )(x2, x2, lrep, len3, W1, b1r, W2, b2r, Wg, bgr, Wseg, bsegr)

    return (logits, labels[:, :, None], scores[:, :, None], cat)


# speedup vs baseline: 1.0184x; 1.0184x over previous
import jax
import jax.numpy as jnp
from jax.experimental import pallas as pl
from jax.experimental.pallas import tpu as pltpu

_B, _S, _DIN, _E, _G, _C = 16, 4096, 8, 64, 128, 8
_R = _B * _DIN
_CHP = 512
_CH = 512
_NOUT = _S // _CH
_PAD = 0.0


def _h_from_packed(xn2, w1, b1r, w2, b2r):
    xs = jnp.concatenate(
        [jnp.transpose(xn2[_DIN * b:_DIN * (b + 1), :]) for b in range(_B)],
        axis=0)
    h = jnp.maximum(xs @ w1 + b1r, 0.0)
    return jnp.maximum(h @ w2 + b2r, 0.0)


def _prep_phase(x2_ref, lrep_ref, len3_ref, w1_ref, b1r_ref, w2_ref,
                b2r_ref, wg_ref, bgr_ref, mr_s, svr_s, gfv_s):
    f32 = jnp.float32
    x2 = x2_ref[...]
    lrep = lrep_ref[...]
    il = jax.lax.broadcasted_iota(jnp.int32, (_R, _S), 1)
    mf = (il < lrep).astype(f32)
    cnt = jnp.maximum(jnp.sum(mf) * (1.0 / _DIN), 1.0)

    rd = jax.lax.broadcasted_iota(jnp.int32, (_DIN, _R), 1)
    dd = jax.lax.broadcasted_iota(jnp.int32, (_DIN, _R), 0)
    m8 = (jax.lax.rem(rd, _DIN) == dd).astype(f32)
    rr = jax.lax.broadcasted_iota(jnp.int32, (_R, _DIN), 0)
    dc = jax.lax.broadcasted_iota(jnp.int32, (_R, _DIN), 1)
    p8 = (jax.lax.rem(rr, _DIN) == dc).astype(f32)

    hi = jax.lax.Precision.HIGHEST
    dn = (((1,), (0,)), ((), ()))
    sum_rows = jnp.sum(x2 * mf, axis=1, keepdims=True)
    mean8 = jax.lax.dot_general(m8, sum_rows, dn, precision=hi) / cnt
    mean_r = jax.lax.dot_general(p8, mean8, dn, precision=hi)
    xc = x2 - mean_r
    sq_rows = jnp.sum((xc * xc) * mf, axis=1, keepdims=True)
    var8 = jax.lax.dot_general(m8, sq_rows, dn, precision=hi) / cnt
    sv8 = jnp.sqrt(var8 + 1e-6)
    sv_r = jax.lax.dot_general(p8, sv8, dn, precision=hi)
    mr_s[...] = mean_r
    svr_s[...] = sv_r
    xn2 = xc / sv_r

    len3 = len3_ref[...]
    w1 = w1_ref[...]
    b1r = b1r_ref[...]
    w2 = w2_ref[...]
    b2r = b2r_ref[...]
    wg = wg_ref[...]
    bgr = bgr_ref[...]
    leng = jnp.broadcast_to(len3, (_B, _CHP, _G))
    igc = jax.lax.broadcasted_iota(jnp.int32, (_B, _CHP, _G), 1)
    gfv = jnp.full((_B, 1, _G), -jnp.inf, dtype=f32)
    for c in range(_S // _CHP):
        h = _h_from_packed(xn2[:, _CHP * c:_CHP * (c + 1)],
                           w1, b1r, w2, b2r)
        g = jnp.maximum(h @ wg + bgr, 0.0)
        maskg = (igc + _CHP * c) < leng
        g3 = jnp.where(maskg, g.reshape(_B, _CHP, _G), -jnp.inf)
        gfv = jnp.maximum(gfv, jnp.max(g3, axis=1, keepdims=True))
    gfv_s[...] = gfv


def _out_phase(k, x2c_ref, len3_ref, wseg_ref, bsegr_ref, w1_ref, b1r_ref,
               w2_ref, b2r_ref, logits_ref, labels_ref, scores_ref,
               cat_ref, mr_s, svr_s, gfv_s):
    base = (k - 1) * _CH
    len3 = len3_ref[...]

    n = _B * _CH
    xn2c = (x2c_ref[...] - mr_s[...]) / svr_s[...]
    h = _h_from_packed(xn2c, w1_ref[...], b1r_ref[...],
                       w2_ref[...], b2r_ref[...])

    ie = jax.lax.broadcasted_iota(jnp.int32, (_B, _CH, _E), 1) + base
    maske = ie < jnp.broadcast_to(len3, (_B, _CH, _E))
    h3 = jnp.where(maske, h.reshape(_B, _CH, _E), _PAD)

    ig = jax.lax.broadcasted_iota(jnp.int32, (_B, _CH, _G), 1) + base
    maskg = ig < jnp.broadcast_to(len3, (_B, _CH, _G))
    gfv3 = jnp.broadcast_to(gfv_s[...], (_B, _CH, _G))
    gfv3 = jnp.where(maskg, gfv3, _PAD)

    cat3 = jnp.concatenate([h3, gfv3], axis=2)
    cat_ref[...] = cat3

    logits = cat3.reshape(n, _E + _G) @ wseg_ref[...] + bsegr_ref[...]
    ic = jax.lax.broadcasted_iota(jnp.int32, (_B, _CH, _C), 1) + base
    maskc = ic < jnp.broadcast_to(len3, (_B, _CH, _C))
    logits2 = jnp.where(maskc.reshape(n, _C), logits, _PAD)
    logits_ref[...] = logits2.reshape(_B, _CH, _C)

    lt = jnp.transpose(logits2)
    m = jnp.max(lt, axis=0, keepdims=True)
    ssum = jnp.sum(jnp.exp(lt - m), axis=0, keepdims=True)
    scores = 1.0 / ssum

    cidx = jax.lax.broadcasted_iota(jnp.int32, (_C, n), 0)
    cand = jnp.where(lt == m, cidx, _C)
    labels = jnp.min(cand, axis=0, keepdims=True)
    labels = jnp.where(jnp.isnan(scores), -1, labels)

    def _rows(v):
        return jnp.concatenate(
            [v[:, _CH * b:_CH * (b + 1)] for b in range(_B)], axis=0)

    scores_ref[...] = _rows(scores)
    labels_ref[...] = _rows(labels)


def _fused_kernel(x2_ref, x2c_ref, lrep_ref, len3_ref, w1_ref, b1r_ref,
                  w2_ref, b2r_ref, wg_ref, bgr_ref, wseg_ref, bsegr_ref,
                  logits_ref, labels_ref, scores_ref, cat_ref,
                  mr_s, svr_s, gfv_s):
    k = pl.program_id(0)

    @pl.when(k == 0)
    def _():
        _prep_phase(x2_ref, lrep_ref, len3_ref, w1_ref, b1r_ref, w2_ref,
                    b2r_ref, wg_ref, bgr_ref, mr_s, svr_s, gfv_s)

    @pl.when(k > 0)
    def _():
        _out_phase(k, x2c_ref, len3_ref, wseg_ref, bsegr_ref, w1_ref,
                   b1r_ref, w2_ref, b2r_ref, logits_ref, labels_ref,
                   scores_ref, cat_ref, mr_s, svr_s, gfv_s)


def kernel(x, lengths, W1, b1, W2, b2, Wg, bg, Wseg, bseg):
    f32 = jnp.float32
    x2 = x.transpose(0, 2, 1).reshape(_R, _S)
    lrep = jnp.repeat(lengths.astype(jnp.int32), _DIN).reshape(_R, 1)
    len3 = lengths.astype(jnp.int32).reshape(_B, 1, 1)
    b1r = b1.reshape(1, _E)
    b2r = b2.reshape(1, _E)
    bgr = bg.reshape(1, _G)
    bsegr = bseg.reshape(1, _C)

    def _ochunk(k):
        kk = jnp.maximum(k - 1, 0)
        return kk

    logits, labels, scores, cat = pl.pallas_call(
        _fused_kernel,
        grid=(1 + _NOUT,),
        in_specs=[
            pl.BlockSpec((_R, _S), lambda k: (0, 0)),
            pl.BlockSpec((_R, _CH), lambda k: (0, _ochunk(k))),
            pl.BlockSpec((_R, 1), lambda k: (0, 0)),
            pl.BlockSpec((_B, 1, 1), lambda k: (0, 0, 0)),
            pl.BlockSpec((_DIN, _E), lambda k: (0, 0)),
            pl.BlockSpec((1, _E), lambda k: (0, 0)),
            pl.BlockSpec((_E, _E), lambda k: (0, 0)),
            pl.BlockSpec((1, _E), lambda k: (0, 0)),
            pl.BlockSpec((_E, _G), lambda k: (0, 0)),
            pl.BlockSpec((1, _G), lambda k: (0, 0)),
            pl.BlockSpec((_E + _G, _C), lambda k: (0, 0)),
            pl.BlockSpec((1, _C), lambda k: (0, 0)),
        ],
        out_specs=[
            pl.BlockSpec((_B, _CH, _C), lambda k: (0, _ochunk(k), 0)),
            pl.BlockSpec((_B, _CH), lambda k: (0, _ochunk(k))),
            pl.BlockSpec((_B, _CH), lambda k: (0, _ochunk(k))),
            pl.BlockSpec((_B, _CH, _E + _G), lambda k: (0, _ochunk(k), 0)),
        ],
        out_shape=[
            jax.ShapeDtypeStruct((_B, _S, _C), f32),
            jax.ShapeDtypeStruct((_B, _S), jnp.int32),
            jax.ShapeDtypeStruct((_B, _S), f32),
            jax.ShapeDtypeStruct((_B, _S, _E + _G), f32),
        ],
        scratch_shapes=[
            pltpu.VMEM((_R, 1), f32),
            pltpu.VMEM((_R, 1), f32),
            pltpu.VMEM((_B, 1, _G), f32),
        ],
    )(x2, x2, lrep, len3, W1, b1r, W2, b2r, Wg, bgr, Wseg, bsegr)

    return (logits, labels[:, :, None], scores[:, :, None], cat)
